# baseline (speedup 1.0000x reference)
import functools

import jax
import jax.numpy as jnp
from jax import lax
from jax.experimental import pallas as pl
from jax.experimental.pallas import tpu as pltpu
from jax.experimental.pallas import tpu_sc as plsc

VOCAB = 100000
EMB = 64
BATCH = 16384

NC = 2
NS = 16
NW = NC * NS
BPW = BATCH // NW
CH = 128
NCH = BPW // CH


K = 16
NIT = BPW // K


def _sc_gather_body(user_h, item_h, su_h, tu_h, si_h, ti_h,
                    osu, otu, osi, oti, idx_u, idx_i, sem):
    c = lax.axis_index("c")
    s = lax.axis_index("s")
    wid = s * NC + c
    base = wid * BPW
    pltpu.sync_copy(user_h.at[pl.ds(base, BPW)], idx_u)
    pltpu.sync_copy(item_h.at[pl.ds(base, BPW)], idx_i)

    def body(j, carry):
        co = j * K
        cps = []
        vu = idx_u[pl.ds(co, K)]
        vi = idx_i[pl.ds(co, K)]
        for i in range(K):
            ru = vu[i]
            ri = vi[i]
            dst = pl.ds(base + co + i, 1)
            cps.append(pltpu.async_copy(su_h.at[pl.ds(ru, 1)],
                                        osu.at[dst], sem))
            cps.append(pltpu.async_copy(tu_h.at[pl.ds(ru, 1)],
                                        otu.at[dst], sem))
            cps.append(pltpu.async_copy(si_h.at[pl.ds(ri, 1)],
                                        osi.at[dst], sem))
            cps.append(pltpu.async_copy(ti_h.at[pl.ds(ri, 1)],
                                        oti.at[dst], sem))
        for cp in cps:
            cp.wait()
        return carry

    lax.fori_loop(0, NIT, body, 0)


@functools.lru_cache(maxsize=1)
def _sc_gather():
    return pl.kernel(
        _sc_gather_body,
        out_type=tuple(jax.ShapeDtypeStruct((BATCH, EMB), jnp.float32)
                       for _ in range(4)),
        mesh=plsc.VectorSubcoreMesh(core_axis_name="c", subcore_axis_name="s",
                                    num_cores=NC, num_subcores=NS),
        scratch_types=[
            pltpu.VMEM((BPW,), jnp.int32),
            pltpu.VMEM((BPW,), jnp.int32),
            pltpu.SemaphoreType.DMA,
        ],
    )


BLK = 2048


def _mlp_body(su_r, tu_r, si_r, ti_r, W1, b1, W2, b2, Wp, bp, out):
    ue = jnp.maximum(su_r[...], tu_r[...])
    ie = jnp.maximum(si_r[...], ti_r[...])
    h = jnp.concatenate([ue, ie], axis=1)
    h = jnp.dot(h, W1[...], preferred_element_type=jnp.float32) + b1[...]
    h = jnp.maximum(h, 0.0)
    h = jnp.dot(h, W2[...], preferred_element_type=jnp.float32) + b2[...]
    h = jnp.maximum(h, 0.0)
    o = jnp.dot(h, Wp[...], preferred_element_type=jnp.float32) + bp[...]
    out[...] = jax.nn.sigmoid(o)


def _row_spec():
    return pl.BlockSpec((BLK, EMB), lambda i: (i, 0))


def _full_spec(shape):
    return pl.BlockSpec(shape, lambda i: tuple(0 for _ in shape))


_mlp = pl.pallas_call(
    _mlp_body,
    grid=(BATCH // BLK,),
    in_specs=[
        _row_spec(), _row_spec(), _row_spec(), _row_spec(),
        _full_spec((2 * EMB, 128)), _full_spec((1, 128)),
        _full_spec((128, 64)), _full_spec((1, 64)),
        _full_spec((64, 1)), _full_spec((1, 1)),
    ],
    out_specs=pl.BlockSpec((BLK, 1), lambda i: (i, 0)),
    out_shape=jax.ShapeDtypeStruct((BATCH, 1), jnp.float32),
)


@jax.jit
def kernel(x, su_emb, tu_emb, si_emb, ti_emb, W1, b1, W2, b2, Wp, bp):
    x = x.astype(jnp.int32)
    user = x[:, 0]
    item = x[:, 1]
    g_su, g_tu, g_si, g_ti = _sc_gather()(user, item, su_emb, tu_emb,
                                          si_emb, ti_emb)
    out = _mlp(g_su, g_tu, g_si, g_ti,
               W1, b1.reshape(1, -1), W2, b2.reshape(1, -1),
               Wp, bp.reshape(1, 1))
    return out[:, 0]


# speedup vs baseline: 3.9995x; 3.9995x over previous
import functools

import jax
import jax.numpy as jnp
from jax import lax
from jax.experimental import pallas as pl
from jax.experimental.pallas import tpu as pltpu
from jax.experimental.pallas import tpu_sc as plsc

VOCAB = 100000
EMB = 64
BATCH = 16384

NC = 2
NS = 16
NW = NC * NS
BPW = BATCH // NW
CH = 128
NCH = BPW // CH


CBLK = 4000


def _concat_body(a, b, out):
    out[...] = jnp.concatenate((a[...], b[...]), axis=1)


_concat_pair = pl.pallas_call(
    _concat_body,
    grid=(VOCAB // CBLK,),
    in_specs=[
        pl.BlockSpec((CBLK, EMB), lambda i: (i, 0)),
        pl.BlockSpec((CBLK, EMB), lambda i: (i, 0)),
    ],
    out_specs=pl.BlockSpec((CBLK, 2 * EMB), lambda i: (i, 0)),
    out_shape=jax.ShapeDtypeStruct((VOCAB, 2 * EMB), jnp.float32),
)


def _sc_gather_body(user_h, item_h, ut_h, it_h, ou, oi,
                    idx_u, idx_i, rows, sem):
    c = lax.axis_index("c")
    s = lax.axis_index("s")
    wid = s * NC + c
    base = wid * BPW
    for j in range(NCH):
        pltpu.sync_copy(user_h.at[pl.ds(base + j * CH, CH)], idx_u.at[j])
        pltpu.sync_copy(item_h.at[pl.ds(base + j * CH, CH)], idx_i.at[j])
    for tbl, out, idx in ((ut_h, ou, idx_u), (it_h, oi, idx_i)):
        cps = [pltpu.async_copy(tbl.at[idx.at[j]],
                                rows.at[pl.ds(j * CH, CH)], sem)
               for j in range(NCH)]
        for cp in cps:
            cp.wait()
        pltpu.sync_copy(rows, out.at[pl.ds(base, BPW)])


@functools.lru_cache(maxsize=1)
def _sc_gather():
    return pl.kernel(
        _sc_gather_body,
        out_type=tuple(jax.ShapeDtypeStruct((BATCH, 2 * EMB), jnp.float32)
                       for _ in range(2)),
        mesh=plsc.VectorSubcoreMesh(core_axis_name="c", subcore_axis_name="s",
                                    num_cores=NC, num_subcores=NS),
        scratch_types=[
            pltpu.VMEM((NCH, CH), jnp.int32),
            pltpu.VMEM((NCH, CH), jnp.int32),
            pltpu.VMEM((BPW, 2 * EMB), jnp.float32),
            pltpu.SemaphoreType.DMA,
        ],
        compiler_params=pltpu.CompilerParams(use_tc_tiling_on_sc=False),
    )


BLK = 2048


def _mlp_body(gu, gi, W1, b1, W2, b2, Wp, bp, out):
    ue = jnp.maximum(gu[:, :EMB], gu[:, EMB:])
    ie = jnp.maximum(gi[:, :EMB], gi[:, EMB:])
    h = jnp.concatenate((ue, ie), axis=1)
    h = jnp.dot(h, W1[...], preferred_element_type=jnp.float32) + b1[...]
    h = jnp.maximum(h, 0.0)
    h = jnp.dot(h, W2[...], preferred_element_type=jnp.float32) + b2[...]
    h = jnp.maximum(h, 0.0)
    o = jnp.dot(h, Wp[...], preferred_element_type=jnp.float32) + bp[...]
    out[...] = jax.nn.sigmoid(o)


def _row_spec():
    return pl.BlockSpec((BLK, 2 * EMB), lambda i: (i, 0))


def _full_spec(shape):
    return pl.BlockSpec(shape, lambda i: tuple(0 for _ in shape))


_mlp = pl.pallas_call(
    _mlp_body,
    grid=(BATCH // BLK,),
    in_specs=[
        _row_spec(), _row_spec(),
        _full_spec((2 * EMB, 128)), _full_spec((1, 128)),
        _full_spec((128, 64)), _full_spec((1, 64)),
        _full_spec((64, 1)), _full_spec((1, 1)),
    ],
    out_specs=pl.BlockSpec((BLK, 1), lambda i: (i, 0)),
    out_shape=jax.ShapeDtypeStruct((BATCH, 1), jnp.float32),
)


@jax.jit
def kernel(x, su_emb, tu_emb, si_emb, ti_emb, W1, b1, W2, b2, Wp, bp):
    x = x.astype(jnp.int32)
    user = x[:, 0]
    item = x[:, 1]
    ut = _concat_pair(su_emb, tu_emb)
    it = _concat_pair(si_emb, ti_emb)
    gu, gi = _sc_gather()(user, item, ut, it)
    out = _mlp(gu, gi,
               W1, b1.reshape(1, -1), W2, b2.reshape(1, -1),
               Wp, bp.reshape(1, 1))
    return out[:, 0]


# speedup vs baseline: 5.1768x; 1.2944x over previous
import functools

import jax
import jax.numpy as jnp
from jax import lax
from jax.experimental import pallas as pl
from jax.experimental.pallas import tpu as pltpu
from jax.experimental.pallas import tpu_sc as plsc

VOCAB = 100000
EMB = 64
BATCH = 16384

NC = 2
NS = 16
NW = NC * NS
BPW = BATCH // NW
CH = 128
NCH = BPW // CH


def _sc_gather_body(user_h, item_h, ut_h, it_h, ou, oi,
                    idx_u, idx_i, rows, sem):
    c = lax.axis_index("c")
    s = lax.axis_index("s")
    wid = s * NC + c
    base = wid * BPW
    for j in range(NCH):
        pltpu.sync_copy(user_h.at[pl.ds(base + j * CH, CH)], idx_u.at[j])
        pltpu.sync_copy(item_h.at[pl.ds(base + j * CH, CH)], idx_i.at[j])
    for tbl, out, idx in ((ut_h, ou, idx_u), (it_h, oi, idx_i)):
        cps = [pltpu.async_copy(tbl.at[idx.at[j]],
                                rows.at[pl.ds(j * CH, CH)], sem)
               for j in range(NCH)]
        for cp in cps:
            cp.wait()
        pltpu.sync_copy(rows, out.at[pl.ds(base, BPW)])


@functools.lru_cache(maxsize=1)
def _sc_gather():
    return pl.kernel(
        _sc_gather_body,
        out_type=tuple(jax.ShapeDtypeStruct((BATCH, 2 * EMB), jnp.float32)
                       for _ in range(2)),
        mesh=plsc.VectorSubcoreMesh(core_axis_name="c", subcore_axis_name="s",
                                    num_cores=NC, num_subcores=NS),
        scratch_types=[
            pltpu.VMEM((NCH, CH), jnp.int32),
            pltpu.VMEM((NCH, CH), jnp.int32),
            pltpu.VMEM((BPW, 2 * EMB), jnp.float32),
            pltpu.SemaphoreType.DMA,
        ],
        compiler_params=pltpu.CompilerParams(use_tc_tiling_on_sc=False),
    )


BLK = 2048


def _mlp_body(gu, gi, W1, b1, W2, b2, Wp, bp, out):
    ue = jnp.maximum(gu[:, :EMB], gu[:, EMB:])
    ie = jnp.maximum(gi[:, :EMB], gi[:, EMB:])
    h = jnp.concatenate((ue, ie), axis=1)
    h = jnp.dot(h, W1[...], preferred_element_type=jnp.float32) + b1[...]
    h = jnp.maximum(h, 0.0)
    h = jnp.dot(h, W2[...], preferred_element_type=jnp.float32) + b2[...]
    h = jnp.maximum(h, 0.0)
    o = jnp.dot(h, Wp[...], preferred_element_type=jnp.float32) + bp[...]
    out[...] = jax.nn.sigmoid(o)


def _row_spec():
    return pl.BlockSpec((BLK, 2 * EMB), lambda i: (i, 0))


def _full_spec(shape):
    return pl.BlockSpec(shape, lambda i: tuple(0 for _ in shape))


_mlp = pl.pallas_call(
    _mlp_body,
    grid=(BATCH // BLK,),
    in_specs=[
        _row_spec(), _row_spec(),
        _full_spec((2 * EMB, 128)), _full_spec((1, 128)),
        _full_spec((128, 64)), _full_spec((1, 64)),
        _full_spec((64, 1)), _full_spec((1, 1)),
    ],
    out_specs=pl.BlockSpec((BLK, 1), lambda i: (i, 0)),
    out_shape=jax.ShapeDtypeStruct((BATCH, 1), jnp.float32),
)


@jax.jit
def kernel(x, su_emb, tu_emb, si_emb, ti_emb, W1, b1, W2, b2, Wp, bp):
    x = x.astype(jnp.int32)
    user = x[:, 0]
    item = x[:, 1]
    ut = jnp.concatenate((su_emb, tu_emb), axis=1)
    it = jnp.concatenate((si_emb, ti_emb), axis=1)
    gu, gi = _sc_gather()(user, item, ut, it)
    out = _mlp(gu, gi,
               W1, b1.reshape(1, -1), W2, b2.reshape(1, -1),
               Wp, bp.reshape(1, 1))
    return out[:, 0]
